# baseline (speedup 1.0000x reference)
import functools

import jax
import jax.numpy as jnp
from jax import lax
from jax.experimental import pallas as pl
from jax.experimental.pallas import tpu as pltpu
from jax.experimental.pallas import tpu_sc as plsc

_N = 10000
_E = 320000
_D = 128
_H = 128
_O = 32
_B = 128
_L = 5

_NC = 2
_NS = 16
_NW = _NC * _NS
_EPT = _E // _NW
_CH = 100
_NCH = _EPT // _CH
_NBUF = 4
_NP = 10240
_RPT = _NP // _NS

_sc_mesh = plsc.VectorSubcoreMesh(core_axis_name="c", subcore_axis_name="s")
_sc_params = pltpu.CompilerParams(use_tc_tiling_on_sc=False)


@functools.partial(
    pl.kernel,
    out_type=jax.ShapeDtypeStruct((_NC, _NP, 16), jnp.float32),
    mesh=_sc_mesh,
    scratch_types=[
        pltpu.VMEM((_NCH, _CH), jnp.int32),
        pltpu.VMEM((_CH, 16), jnp.float32),
        pltpu.VMEM_SHARED((_NP, 16), jnp.float32),
    ],
    compiler_params=_sc_params,
)
def _deg_kernel(row_hbm, ones_hbm, zeros_hbm, out_hbm, rowv, onesv, degacc):
    c = lax.axis_index("c")
    s = lax.axis_index("s")
    wid = c * _NS + s
    pltpu.sync_copy(row_hbm.at[wid], rowv)
    pltpu.sync_copy(ones_hbm, onesv)
    pltpu.sync_copy(zeros_hbm, degacc.at[pl.ds(s * _RPT, _RPT)])
    plsc.subcore_barrier()

    def chunk(ch, carry):
        pltpu.sync_copy(onesv, degacc.at[rowv.at[ch]], add=True)
        return carry

    lax.fori_loop(0, _NCH, chunk, 0)
    plsc.subcore_barrier()
    pltpu.sync_copy(degacc.at[pl.ds(s * _RPT, _RPT)],
                    out_hbm.at[c, pl.ds(s * _RPT, _RPT)])


@functools.partial(
    pl.kernel,
    out_type=jax.ShapeDtypeStruct((_NC, _NP, _H), jnp.bfloat16),
    mesh=_sc_mesh,
    scratch_types=[
        pltpu.VMEM((_NCH, _CH), jnp.int32),
        pltpu.VMEM((_NCH, _CH), jnp.int32),
        [pltpu.VMEM((_CH, _H), jnp.bfloat16)] * _NBUF,
        [pltpu.SemaphoreType.DMA] * _NBUF,
        [pltpu.SemaphoreType.DMA] * _NBUF,
        pltpu.VMEM_SHARED((_NP, _H), jnp.bfloat16),
    ],
    compiler_params=_sc_params,
)
def _spmm_kernel(hp_hbm, row_hbm, col_hbm, zeros_hbm, out_hbm,
                 rowv, colv, gbufs, gsems, ssems, acc):
    c = lax.axis_index("c")
    s = lax.axis_index("s")
    wid = c * _NS + s
    pltpu.sync_copy(row_hbm.at[wid], rowv)
    pltpu.sync_copy(col_hbm.at[wid], colv)
    pltpu.sync_copy(zeros_hbm, acc.at[pl.ds(s * _RPT, _RPT)])
    plsc.subcore_barrier()

    def group(g, carry):
        for k in range(_NBUF):
            ch = g * _NBUF + k

            @pl.when(g > 0)
            def _():
                pltpu.make_async_copy(
                    gbufs[k], acc.at[rowv.at[ch - _NBUF]], ssems[k]).wait()

            pltpu.async_copy(hp_hbm.at[colv.at[ch]], gbufs[k], gsems[k])
        for k in range(_NBUF):
            ch = g * _NBUF + k
            pltpu.make_async_copy(
                hp_hbm.at[colv.at[ch]], gbufs[k], gsems[k]).wait()
            pltpu.async_copy(gbufs[k], acc.at[rowv.at[ch]], ssems[k],
                             add=True)
        return carry

    lax.fori_loop(0, _NCH // _NBUF, group, 0)
    for k in range(_NBUF):
        pltpu.make_async_copy(
            gbufs[k], acc.at[rowv.at[_NCH - _NBUF + k]], ssems[k]).wait()
    plsc.subcore_barrier()
    pltpu.sync_copy(acc.at[pl.ds(s * _RPT, _RPT)],
                    out_hbm.at[c, pl.ds(s * _RPT, _RPT)])


def _dot(a, b):
    return jnp.dot(a, b, preferred_element_type=jnp.float32,
                   precision=lax.Precision.DEFAULT)


def _prep_body(deg_ref, x_ref, p0_ref, dis_ref, hp_ref, hpb_ref, z_ref):
    deg = deg_ref[...]
    dis = lax.rsqrt(jnp.maximum(deg, 1.0))
    dis_ref[...] = dis
    hp = dis * x_ref[...]
    hp_ref[...] = hp
    hpb_ref[...] = hp.astype(jnp.bfloat16)
    z_ref[...] = _dot(x_ref[...], p0_ref[...])


_prep = pl.pallas_call(
    _prep_body,
    out_shape=(
        jax.ShapeDtypeStruct((_N, 1), jnp.float32),
        jax.ShapeDtypeStruct((_N, _D), jnp.float32),
        jax.ShapeDtypeStruct((_N, _D), jnp.bfloat16),
        jax.ShapeDtypeStruct((_N, _O), jnp.float32),
    ),
)


def _bn_relu(acc_ref, hp_ref, dis_ref, w_ref, b_ref, g_ref, be_ref):
    dis = dis_ref[...]
    m = dis * (acc_ref[...] + hp_ref[...])
    t = _dot(m, w_ref[...]) + b_ref[...]
    mu = jnp.mean(t, axis=0, keepdims=True)
    var = jnp.mean(jnp.square(t - mu), axis=0, keepdims=True)
    h = (t - mu) * lax.rsqrt(var + 1e-5) * g_ref[...] + be_ref[...]
    return jnp.maximum(h, 0.0), dis


def _layer_body(acc_ref, hp_ref, dis_ref, w_ref, b_ref, g_ref, be_ref,
                p_ref, zin_ref, hpo_ref, hpob_ref, zo_ref):
    h, dis = _bn_relu(acc_ref, hp_ref, dis_ref, w_ref, b_ref, g_ref, be_ref)
    hp = dis * h
    hpo_ref[...] = hp
    hpob_ref[...] = hp.astype(jnp.bfloat16)
    zo_ref[...] = zin_ref[...] + _dot(h, p_ref[...])


_layer = pl.pallas_call(
    _layer_body,
    out_shape=(
        jax.ShapeDtypeStruct((_N, _H), jnp.float32),
        jax.ShapeDtypeStruct((_N, _H), jnp.bfloat16),
        jax.ShapeDtypeStruct((_N, _O), jnp.float32),
    ),
)


def _final_body(acc_ref, hp_ref, dis_ref, w_ref, b_ref, g_ref, be_ref,
                p_ref, zin_ref, gid_ref, pbt_ref, score_ref):
    h, _ = _bn_relu(acc_ref, hp_ref, dis_ref, w_ref, b_ref, g_ref, be_ref)
    z = zin_ref[...] + _dot(h, p_ref[...])
    onehot = (lax.broadcasted_iota(jnp.int32, (_B, _N), 0)
              == gid_ref[...]).astype(jnp.float32)
    score_ref[...] = _dot(onehot, z) + pbt_ref[...]


_final = pl.pallas_call(
    _final_body,
    out_shape=jax.ShapeDtypeStruct((_B, _O), jnp.float32),
)


def kernel(x, params, edge_index, graph_ids):
    row2d = edge_index[0].reshape(_NW, _NCH, _CH)
    col2d = edge_index[1].reshape(_NW, _NCH, _CH)
    ones16 = jnp.ones((_CH, 16), jnp.float32)
    zeros16 = jnp.zeros((_RPT, 16), jnp.float32)
    zerosH = jnp.zeros((_RPT, _H), jnp.bfloat16)
    pbt = (params["pb0"] + params["pb1"] + params["pb2"] + params["pb3"]
           + params["pb4"])[None, :]

    deg_parts = _deg_kernel(row2d, ones16, zeros16)
    deg = (deg_parts[0, :_N, 0:1] + deg_parts[1, :_N, 0:1]) + 1.0
    dis, hp, hpb, z = _prep(deg, x, params["P0"])
    for l in range(_L - 1):
        acc = _spmm_kernel(hpb, row2d, col2d, zerosH)
        acc = (acc[0, :_N] + acc[1, :_N]).astype(jnp.float32)
        w = params["W%d" % l]
        b = params["b%d" % l][None, :]
        g = params["g%d" % l][None, :]
        be = params["be%d" % l][None, :]
        p = params["P%d" % (l + 1)]
        if l < _L - 2:
            hp, hpb, z = _layer(acc, hp, dis, w, b, g, be, p, z)
        else:
            score = _final(acc, hp, dis, w, b, g, be, p, z,
                           graph_ids[None, :], pbt)
    return score


# speedup vs baseline: 1.0659x; 1.0659x over previous
import functools

import jax
import jax.numpy as jnp
from jax import lax
from jax.experimental import pallas as pl
from jax.experimental.pallas import tpu as pltpu
from jax.experimental.pallas import tpu_sc as plsc

_N = 10000
_E = 320000
_D = 128
_H = 128
_O = 32
_B = 128
_L = 5

_NC = 2
_NS = 16
_NW = _NC * _NS
_EPT = _E // _NW
_CH = 100
_NCH = _EPT // _CH
_NBUF = 8
_NP = 10240
_RPT = _NP // _NS

_sc_mesh = plsc.VectorSubcoreMesh(core_axis_name="c", subcore_axis_name="s")
_sc_params = pltpu.CompilerParams(use_tc_tiling_on_sc=False)


@functools.partial(
    pl.kernel,
    out_type=jax.ShapeDtypeStruct((_NC, _NP, 16), jnp.float32),
    mesh=_sc_mesh,
    scratch_types=[
        pltpu.VMEM((_NCH, _CH), jnp.int32),
        pltpu.VMEM((_CH, 16), jnp.float32),
        pltpu.VMEM_SHARED((_NP, 16), jnp.float32),
    ],
    compiler_params=_sc_params,
)
def _deg_kernel(row_hbm, ones_hbm, zeros_hbm, out_hbm, rowv, onesv, degacc):
    c = lax.axis_index("c")
    s = lax.axis_index("s")
    wid = c * _NS + s
    pltpu.sync_copy(row_hbm.at[wid], rowv)
    pltpu.sync_copy(ones_hbm, onesv)
    pltpu.sync_copy(zeros_hbm, degacc.at[pl.ds(s * _RPT, _RPT)])
    plsc.subcore_barrier()

    def chunk(ch, carry):
        pltpu.sync_copy(onesv, degacc.at[rowv.at[ch]], add=True)
        return carry

    lax.fori_loop(0, _NCH, chunk, 0)
    plsc.subcore_barrier()
    pltpu.sync_copy(degacc.at[pl.ds(s * _RPT, _RPT)],
                    out_hbm.at[c, pl.ds(s * _RPT, _RPT)])


@functools.partial(
    pl.kernel,
    out_type=jax.ShapeDtypeStruct((_NC, _NP, _H), jnp.bfloat16),
    mesh=_sc_mesh,
    scratch_types=[
        pltpu.VMEM((_NCH, _CH), jnp.int32),
        pltpu.VMEM((_NCH, _CH), jnp.int32),
        [pltpu.VMEM((_CH, _H), jnp.bfloat16)] * _NBUF,
        [pltpu.SemaphoreType.DMA] * _NBUF,
        [pltpu.SemaphoreType.DMA] * _NBUF,
        pltpu.VMEM_SHARED((_NP, _H), jnp.bfloat16),
    ],
    compiler_params=_sc_params,
)
def _spmm_kernel(hp_hbm, row_hbm, col_hbm, zeros_hbm, out_hbm,
                 rowv, colv, gbufs, gsems, ssems, acc):
    c = lax.axis_index("c")
    s = lax.axis_index("s")
    wid = c * _NS + s
    pltpu.sync_copy(row_hbm.at[wid], rowv)
    pltpu.sync_copy(col_hbm.at[wid], colv)
    pltpu.sync_copy(zeros_hbm, acc.at[pl.ds(s * _RPT, _RPT)])
    plsc.subcore_barrier()

    def group(g, carry):
        for k in range(_NBUF):
            ch = g * _NBUF + k

            @pl.when(g > 0)
            def _():
                pltpu.make_async_copy(
                    gbufs[k], acc.at[rowv.at[ch - _NBUF]], ssems[k]).wait()

            pltpu.async_copy(hp_hbm.at[colv.at[ch]], gbufs[k], gsems[k])
        for k in range(_NBUF):
            ch = g * _NBUF + k
            pltpu.make_async_copy(
                hp_hbm.at[colv.at[ch]], gbufs[k], gsems[k]).wait()
            pltpu.async_copy(gbufs[k], acc.at[rowv.at[ch]], ssems[k],
                             add=True)
        return carry

    lax.fori_loop(0, _NCH // _NBUF, group, 0)
    for k in range(_NBUF):
        pltpu.make_async_copy(
            gbufs[k], acc.at[rowv.at[_NCH - _NBUF + k]], ssems[k]).wait()
    plsc.subcore_barrier()
    pltpu.sync_copy(acc.at[pl.ds(s * _RPT, _RPT)],
                    out_hbm.at[c, pl.ds(s * _RPT, _RPT)])


def _dot(a, b):
    return jnp.dot(a, b, preferred_element_type=jnp.float32,
                   precision=lax.Precision.DEFAULT)


def _prep_body(deg_ref, x_ref, p0_ref, dis_ref, hpb_ref, z_ref):
    deg = deg_ref[...]
    dis = lax.rsqrt(jnp.maximum(deg, 1.0))
    dis_ref[...] = dis
    hpb_ref[...] = (dis * x_ref[...]).astype(jnp.bfloat16)
    z_ref[...] = _dot(x_ref[...], p0_ref[...])


_prep = pl.pallas_call(
    _prep_body,
    out_shape=(
        jax.ShapeDtypeStruct((_N, 1), jnp.float32),
        jax.ShapeDtypeStruct((_N, _D), jnp.bfloat16),
        jax.ShapeDtypeStruct((_N, _O), jnp.float32),
    ),
)


def _bn_relu(acc_ref, hpb_ref, dis_ref, w_ref, b_ref, g_ref, be_ref):
    dis = dis_ref[...]
    m = dis * (acc_ref[...] + hpb_ref[...].astype(jnp.float32))
    t = _dot(m, w_ref[...]) + b_ref[...]
    mu = jnp.mean(t, axis=0, keepdims=True)
    var = jnp.mean(jnp.square(t - mu), axis=0, keepdims=True)
    h = (t - mu) * lax.rsqrt(var + 1e-5) * g_ref[...] + be_ref[...]
    return jnp.maximum(h, 0.0), dis


def _layer_body(acc_ref, hpb_ref, dis_ref, w_ref, b_ref, g_ref, be_ref,
                p_ref, zin_ref, hpob_ref, zo_ref):
    h, dis = _bn_relu(acc_ref, hpb_ref, dis_ref, w_ref, b_ref, g_ref, be_ref)
    hpob_ref[...] = (dis * h).astype(jnp.bfloat16)
    zo_ref[...] = zin_ref[...] + _dot(h, p_ref[...])


_layer = pl.pallas_call(
    _layer_body,
    out_shape=(
        jax.ShapeDtypeStruct((_N, _H), jnp.bfloat16),
        jax.ShapeDtypeStruct((_N, _O), jnp.float32),
    ),
)


def _final_body(acc_ref, hpb_ref, dis_ref, w_ref, b_ref, g_ref, be_ref,
                p_ref, zin_ref, gid_ref, pbt_ref, score_ref):
    h, _ = _bn_relu(acc_ref, hpb_ref, dis_ref, w_ref, b_ref, g_ref, be_ref)
    z = zin_ref[...] + _dot(h, p_ref[...])
    onehot = (lax.broadcasted_iota(jnp.int32, (_B, _N), 0)
              == gid_ref[...]).astype(jnp.float32)
    score_ref[...] = _dot(onehot, z) + pbt_ref[...]


_final = pl.pallas_call(
    _final_body,
    out_shape=jax.ShapeDtypeStruct((_B, _O), jnp.float32),
)


def kernel(x, params, edge_index, graph_ids):
    row2d = edge_index[0].reshape(_NW, _NCH, _CH)
    col2d = edge_index[1].reshape(_NW, _NCH, _CH)
    ones16 = jnp.ones((_CH, 16), jnp.float32)
    zeros16 = jnp.zeros((_RPT, 16), jnp.float32)
    zerosH = jnp.zeros((_RPT, _H), jnp.bfloat16)
    pbt = (params["pb0"] + params["pb1"] + params["pb2"] + params["pb3"]
           + params["pb4"])[None, :]

    deg_parts = _deg_kernel(row2d, ones16, zeros16)
    deg = (deg_parts[0, :_N, 0:1] + deg_parts[1, :_N, 0:1]) + 1.0
    dis, hpb, z = _prep(deg, x, params["P0"])
    for l in range(_L - 1):
        acc = _spmm_kernel(hpb, row2d, col2d, zerosH)
        acc = (acc[0, :_N] + acc[1, :_N]).astype(jnp.float32)
        w = params["W%d" % l]
        b = params["b%d" % l][None, :]
        g = params["g%d" % l][None, :]
        be = params["be%d" % l][None, :]
        p = params["P%d" % (l + 1)]
        if l < _L - 2:
            hpb, z = _layer(acc, hpb, dis, w, b, g, be, p, z)
        else:
            score = _final(acc, hpb, dis, w, b, g, be, p, z,
                           graph_ids[None, :], pbt)
    return score
